# baseline (speedup 1.0000x reference)
import functools

import jax
import jax.numpy as jnp
from jax import lax
from jax.experimental import pallas as pl
from jax.experimental.pallas import tpu as pltpu
from jax.experimental.pallas import tpu_sc as plsc

B = 16384
N_DENSE = 13
NF = 26
VOCAB = 100000
ED = 16
NCOL = N_DENSE + NF

VB = 1024
NVB = 98
VOCAB_PAD = VB * NVB
S_LEN = NF * VOCAB_PAD

NC, NS, L = 2, 16, 16
NW = NC * NS
RPW = B // NW
NIDX = RPW * NF
SEG = 128
NSEG = NIDX // SEG
GRP = RPW // L


def _rowsum_body(x_ref, o_ref):
  o_ref[...] = jnp.sum(x_ref[0], axis=0)


def _rowsum(emb_t):
  return pl.pallas_call(
      _rowsum_body,
      grid=(NF, NVB),
      in_specs=[pl.BlockSpec((1, ED, VB), lambda f, v: (f, 0, v))],
      out_specs=pl.BlockSpec((VB,), lambda f, v: (f * NVB + v,)),
      out_shape=jax.ShapeDtypeStruct((S_LEN,), jnp.float32),
  )(emb_t)


def _sc_body(inp_hbm, s_hbm, w_hbm, bias_hbm, out_hbm,
             inp_v, idx_v, gbuf, dbuf, outb, wv, bv, sem):
  wid = lax.axis_index("s") * NC + lax.axis_index("c")
  base = wid * RPW
  pltpu.sync_copy(inp_hbm.at[pl.ds(base, RPW)], inp_v)
  pltpu.sync_copy(w_hbm, wv)
  pltpu.sync_copy(bias_hbm, bv)

  iota = lax.iota(jnp.int32, L)
  offs_a = iota * VOCAB_PAD
  offs_b = (iota + 10) * VOCAB_PAD

  def build_row(j, carry):
    a = inp_v[j, pl.ds(N_DENSE, L)].astype(jnp.int32) + offs_a
    b = inp_v[j, pl.ds(NCOL - L, L)].astype(jnp.int32) + offs_b
    idx_v[pl.ds(j * NF, L)] = a
    idx_v[pl.ds(j * NF + (NF - L), L)] = b
    return carry

  lax.fori_loop(0, RPW, build_row, 0)

  def issue(m, c):
    pltpu.async_copy(s_hbm.at[idx_v.at[pl.ds(m * SEG, SEG)]],
                     gbuf.at[pl.ds(m * SEG, SEG)], sem)
    return c

  lax.fori_loop(0, NSEG, issue, 0)

  wvec = wv[...]
  bvec = bv[...]

  def dense_stage(g, c):
    def drow(i, c2):
      dbuf[i, :] = inp_v[g * L + i, pl.ds(0, L)] * wvec
      return c2
    lax.fori_loop(0, L, drow, 0)
    out_vec = bvec
    for k in range(N_DENSE):
      out_vec = out_vec + plsc.load_gather(dbuf, [iota, iota * 0 + k])
    outb[pl.ds(g * L, L)] = out_vec
    return c

  lax.fori_loop(0, GRP, dense_stage, 0)

  pltpu.make_async_copy(s_hbm.at[pl.ds(0, NIDX)], gbuf, sem).wait()

  def grp_stage(g, c):
    out_vec = outb[pl.ds(g * L, L)]
    gb = g * L * NF
    for p in range(NF):
      out_vec = out_vec + plsc.load_gather(gbuf, [iota * NF + (gb + p)])
    outb[pl.ds(g * L, L)] = out_vec
    return c

  lax.fori_loop(0, GRP, grp_stage, 0)
  pltpu.sync_copy(outb, out_hbm.at[pl.ds(base, RPW)])


def kernel(inputs, emb_tables, dense_weight, bias):
  s_flat = _rowsum(emb_tables.transpose(0, 2, 1))
  w_pad = jnp.concatenate(
      [dense_weight[:, 0], jnp.zeros((L - N_DENSE,), jnp.float32)])
  bias_vec = jnp.broadcast_to(bias, (L,))

  mesh = plsc.VectorSubcoreMesh(core_axis_name="c", subcore_axis_name="s")
  out = pl.kernel(
      _sc_body,
      out_type=jax.ShapeDtypeStruct((B,), jnp.float32),
      mesh=mesh,
      compiler_params=pltpu.CompilerParams(
          needs_layout_passes=False, use_tc_tiling_on_sc=False),
      scratch_types=[
          pltpu.VMEM((RPW, NCOL), jnp.float32),
          pltpu.VMEM((NIDX,), jnp.int32),
          pltpu.VMEM((NIDX,), jnp.float32),
          pltpu.VMEM((L, L), jnp.float32),
          pltpu.VMEM((RPW,), jnp.float32),
          pltpu.VMEM((L,), jnp.float32),
          pltpu.VMEM((L,), jnp.float32),
          pltpu.SemaphoreType.DMA,
      ],
  )(inputs, s_flat, w_pad, bias_vec)
  return out[:, None]


# speedup vs baseline: 11.0974x; 11.0974x over previous
import functools

import jax
import jax.numpy as jnp
from jax import lax
from jax.experimental import pallas as pl
from jax.experimental.pallas import tpu as pltpu
from jax.experimental.pallas import tpu_sc as plsc

B = 16384
N_DENSE = 13
NF = 26
VOCAB = 100000
ED = 16
NCOL = N_DENSE + NF

VB = 1024
NVB = 98
VOCAB_PAD = VB * NVB
S_LEN = NF * VOCAB_PAD

NC, NS, L = 2, 16, 16
NW = NC * NS
RPW = B // NW
NIDX = RPW * NF
SEG = 128
NSEG = NIDX // SEG
GRP = RPW // L


def _rowsum_body(x_ref, o_ref):
  o_ref[...] = jnp.sum(x_ref[0], axis=0)


def _rowsum(emb_t):
  return pl.pallas_call(
      _rowsum_body,
      grid=(NF,),
      in_specs=[pl.BlockSpec((1, ED, VOCAB_PAD), lambda f: (f, 0, 0))],
      out_specs=pl.BlockSpec((VOCAB_PAD,), lambda f: (f,)),
      out_shape=jax.ShapeDtypeStruct((S_LEN,), jnp.float32),
  )(emb_t)


def _sc_body(inp_hbm, s_hbm, w_hbm, bias_hbm, out_hbm,
             inp_v, idx_v, gbuf, dbuf, outb, wv, bv, sem):
  wid = lax.axis_index("s") * NC + lax.axis_index("c")
  base = wid * RPW
  pltpu.sync_copy(inp_hbm.at[pl.ds(base, RPW)], inp_v)
  pltpu.sync_copy(w_hbm, wv)
  pltpu.sync_copy(bias_hbm, bv)

  iota = lax.iota(jnp.int32, L)
  offs_a = iota * VOCAB_PAD
  offs_b = (iota + 10) * VOCAB_PAD

  def build_row(j, carry):
    a = inp_v[j, pl.ds(N_DENSE, L)].astype(jnp.int32) + offs_a
    b = inp_v[j, pl.ds(NCOL - L, L)].astype(jnp.int32) + offs_b
    idx_v[pl.ds(j * NF, L)] = a
    idx_v[pl.ds(j * NF + (NF - L), L)] = b
    return carry

  lax.fori_loop(0, RPW, build_row, 0)

  def issue(m, c):
    pltpu.async_copy(s_hbm.at[idx_v.at[pl.ds(m * SEG, SEG)]],
                     gbuf.at[pl.ds(m * SEG, SEG)], sem)
    return c

  lax.fori_loop(0, NSEG, issue, 0)

  wvec = wv[...]
  bvec = bv[...]

  def dense_stage(g, c):
    def drow(i, c2):
      dbuf[i, :] = inp_v[g * L + i, pl.ds(0, L)] * wvec
      return c2
    lax.fori_loop(0, L, drow, 0)
    out_vec = bvec
    for k in range(N_DENSE):
      out_vec = out_vec + plsc.load_gather(dbuf, [iota, iota * 0 + k])
    outb[pl.ds(g * L, L)] = out_vec
    return c

  lax.fori_loop(0, GRP, dense_stage, 0)

  pltpu.make_async_copy(s_hbm.at[pl.ds(0, NIDX)], gbuf, sem).wait()

  def grp_stage(g, c):
    out_vec = outb[pl.ds(g * L, L)]
    gb = g * L * NF
    for p in range(NF):
      out_vec = out_vec + plsc.load_gather(gbuf, [iota * NF + (gb + p)])
    outb[pl.ds(g * L, L)] = out_vec
    return c

  lax.fori_loop(0, GRP, grp_stage, 0)
  pltpu.sync_copy(outb, out_hbm.at[pl.ds(base, RPW)])


def kernel(inputs, emb_tables, dense_weight, bias):
  s_flat = _rowsum(emb_tables.transpose(0, 2, 1))
  w_pad = jnp.concatenate(
      [dense_weight[:, 0], jnp.zeros((L - N_DENSE,), jnp.float32)])
  bias_vec = jnp.broadcast_to(bias, (L,))

  mesh = plsc.VectorSubcoreMesh(core_axis_name="c", subcore_axis_name="s")
  out = pl.kernel(
      _sc_body,
      out_type=jax.ShapeDtypeStruct((B,), jnp.float32),
      mesh=mesh,
      compiler_params=pltpu.CompilerParams(
          needs_layout_passes=False, use_tc_tiling_on_sc=False),
      scratch_types=[
          pltpu.VMEM((RPW, NCOL), jnp.float32),
          pltpu.VMEM((NIDX,), jnp.int32),
          pltpu.VMEM((NIDX,), jnp.float32),
          pltpu.VMEM((L, L), jnp.float32),
          pltpu.VMEM((RPW,), jnp.float32),
          pltpu.VMEM((L,), jnp.float32),
          pltpu.VMEM((L,), jnp.float32),
          pltpu.SemaphoreType.DMA,
      ],
  )(inputs, s_flat, w_pad, bias_vec)
  return out[:, None]


# speedup vs baseline: 13.2833x; 1.1970x over previous
import jax
import jax.numpy as jnp
from jax import lax
from jax.experimental import pallas as pl
from jax.experimental.pallas import tpu as pltpu
from jax.experimental.pallas import tpu_sc as plsc

B = 16384
N_DENSE = 13
NF = 26
VOCAB = 100000
ED = 16
NCOL = N_DENSE + NF

VOCAB_PAD = 98 * 1024
S_LEN = NF * VOCAB_PAD

NC, NS, L = 2, 16, 16
NW = NC * NS
RPW = B // NW
NIDX = RPW * NF
SEG = 128
NSEG_F = RPW // SEG
GRP = RPW // L


def _rowsum_body(x_ref, o_ref):
  o_ref[...] = jnp.sum(x_ref[0], axis=0)


def _rowsum(emb_t):
  return pl.pallas_call(
      _rowsum_body,
      grid=(NF,),
      in_specs=[pl.BlockSpec((1, ED, VOCAB_PAD), lambda f: (f, 0, 0))],
      out_specs=pl.BlockSpec((VOCAB_PAD,), lambda f: (f,)),
      out_shape=jax.ShapeDtypeStruct((S_LEN,), jnp.float32),
  )(emb_t)


def _sc_body(inpt_hbm, s_hbm, w_hbm, bias_hbm, out_hbm,
             inp_v, idx_v, gbuf, outb, wv, bv, sem):
  wid = lax.axis_index("s") * NC + lax.axis_index("c")
  base = wid * RPW
  pltpu.sync_copy(inpt_hbm.at[:, pl.ds(base, RPW)], inp_v)
  pltpu.sync_copy(w_hbm, wv)
  pltpu.sync_copy(bias_hbm, bv)

  def build_field(f, carry):
    off = f * VOCAB_PAD

    def chunk(c, c2):
      idx_v[pl.ds(f * RPW + c * L, L)] = (
          inp_v[N_DENSE + f, pl.ds(c * L, L)].astype(jnp.int32) + off)
      return c2

    lax.fori_loop(0, GRP, chunk, 0)

    def issue(m, c2):
      o = f * RPW + m * SEG
      pltpu.async_copy(s_hbm.at[idx_v.at[pl.ds(o, SEG)]],
                       gbuf.at[pl.ds(o, SEG)], sem)
      return c2

    lax.fori_loop(0, NSEG_F, issue, 0)
    return carry

  lax.fori_loop(0, NF, build_field, 0)

  bvec = bv[...]

  def dense_stage(g, c):
    out_vec = bvec
    for k in range(N_DENSE):
      out_vec = out_vec + inp_v[k, pl.ds(g * L, L)] * wv[k, :]
    outb[pl.ds(g * L, L)] = out_vec
    return c

  lax.fori_loop(0, GRP, dense_stage, 0)

  pltpu.make_async_copy(s_hbm.at[pl.ds(0, NIDX)], gbuf, sem).wait()

  def grp_stage(g, c):
    out_vec = outb[pl.ds(g * L, L)]
    for p in range(NF):
      out_vec = out_vec + gbuf[pl.ds(p * RPW + g * L, L)]
    outb[pl.ds(g * L, L)] = out_vec
    return c

  lax.fori_loop(0, GRP, grp_stage, 0)
  pltpu.sync_copy(outb, out_hbm.at[pl.ds(base, RPW)])


def kernel(inputs, emb_tables, dense_weight, bias):
  s_flat = _rowsum(emb_tables.transpose(0, 2, 1))
  w_bcast = jnp.broadcast_to(dense_weight, (N_DENSE, L))
  bias_vec = jnp.broadcast_to(bias, (L,))

  mesh = plsc.VectorSubcoreMesh(core_axis_name="c", subcore_axis_name="s")
  out = pl.kernel(
      _sc_body,
      out_type=jax.ShapeDtypeStruct((B,), jnp.float32),
      mesh=mesh,
      compiler_params=pltpu.CompilerParams(
          needs_layout_passes=False, use_tc_tiling_on_sc=False),
      scratch_types=[
          pltpu.VMEM((NCOL, RPW), jnp.float32),
          pltpu.VMEM((NIDX,), jnp.int32),
          pltpu.VMEM((NIDX,), jnp.float32),
          pltpu.VMEM((RPW,), jnp.float32),
          pltpu.VMEM((N_DENSE, L), jnp.float32),
          pltpu.VMEM((L,), jnp.float32),
          pltpu.SemaphoreType.DMA,
      ],
  )(inputs.T, s_flat, w_bcast, bias_vec)
  return out[:, None]
